# baseline (speedup 1.0000x reference)
import functools

import jax
import jax.numpy as jnp
from jax.experimental import pallas as pl
from jax.experimental.pallas import tpu as pltpu


def _round_up(x, m):
    return (x + m - 1) // m * m


def _fused_kernel(x_ref, w_ref, b_ref, o_ref, acc_ref, *, hw, ts, inv_hw, need_mask):
    s = pl.program_id(1)

    @pl.when(s == 0)
    def _():
        acc_ref[...] = jnp.zeros_like(acc_ref)

    x = x_ref[0]
    if need_mask:
        pos = s * ts + jax.lax.broadcasted_iota(jnp.int32, x.shape, 1)
        x = jnp.where(pos < hw, x, 0.0)

    acc = acc_ref[...]
    for j in range(ts // 128):
        acc = acc + x[:, j * 128:(j + 1) * 128]
    acc_ref[...] = acc

    @pl.when(s == pl.num_programs(1) - 1)
    def _():
        m = jnp.dot(w_ref[...], acc_ref[...],
                    preferred_element_type=jnp.float32,
                    precision=jax.lax.Precision.HIGHEST)
        y = jnp.sum(m, axis=1, keepdims=True) * inv_hw + b_ref[...]
        o_ref[...] = jnp.broadcast_to(y[None], o_ref.shape)


def kernel(x_nchw, conv_w, conv_b):
    N, Cin, H, W = x_nchw.shape
    Cout = conv_w.shape[0]
    HW = H * W

    x = x_nchw.reshape(N, Cin, HW).astype(jnp.float32)
    w = conv_w.reshape(Cout, Cin).astype(jnp.float32)
    b = conv_b.reshape(Cout, 1).astype(jnp.float32)

    hw_pad = _round_up(HW, 128)
    ts = min(hw_pad, 512)
    n_tiles = pl.cdiv(HW, ts)
    need_mask = HW % ts != 0

    vmem = int(min(48 << 20,
                   2 * Cin * ts * 4
                   + 2 * Cout * Cin * 4
                   + 2 * Cout * HW * 4
                   + _round_up(Cin, 8) * 128 * 4
                   + (8 << 20)))

    out = pl.pallas_call(
        functools.partial(_fused_kernel, hw=HW, ts=ts, inv_hw=1.0 / HW,
                          need_mask=need_mask),
        out_shape=jax.ShapeDtypeStruct((N, Cout, HW), jnp.float32),
        grid=(N, n_tiles),
        in_specs=[
            pl.BlockSpec((1, Cin, ts), lambda n, s: (n, 0, s)),
            pl.BlockSpec((Cout, Cin), lambda n, s: (0, 0)),
            pl.BlockSpec((Cout, 1), lambda n, s: (0, 0)),
        ],
        out_specs=pl.BlockSpec((1, Cout, HW), lambda n, s: (n, 0, 0)),
        scratch_shapes=[pltpu.VMEM((Cin, 128), jnp.float32)],
        compiler_params=pltpu.CompilerParams(
            dimension_semantics=("parallel", "arbitrary"),
            vmem_limit_bytes=vmem),
    )(x, w, b)

    return out.reshape(N, Cout, H, W)


# speedup vs baseline: 1.1176x; 1.1176x over previous
import functools

import jax
import jax.numpy as jnp
from jax.experimental import pallas as pl
from jax.experimental.pallas import tpu as pltpu


def _round_up(x, m):
    return (x + m - 1) // m * m


def _fused_kernel(x_ref, w_ref, b_ref, o_ref, acc_ref, *, hw, ts, inv_hw, need_mask):
    s = pl.program_id(1)

    @pl.when(s == 0)
    def _():
        acc_ref[...] = jnp.zeros_like(acc_ref)

    x = x_ref[0]
    if need_mask:
        pos = s * ts + jax.lax.broadcasted_iota(jnp.int32, x.shape, 1)
        x = jnp.where(pos < hw, x, 0.0)

    acc = acc_ref[...]
    for j in range(ts // 128):
        acc = acc + x[:, j * 128:(j + 1) * 128]
    acc_ref[...] = acc

    @pl.when(s == pl.num_programs(1) - 1)
    def _():
        m = jnp.dot(w_ref[...], acc_ref[...],
                    preferred_element_type=jnp.float32,
                    precision=jax.lax.Precision.HIGHEST)
        y = jnp.sum(m, axis=1, keepdims=True) * inv_hw + b_ref[...]
        o_ref[...] = jnp.broadcast_to(y[None], o_ref.shape)


def kernel(x_nchw, conv_w, conv_b):
    N, Cin, H, W = x_nchw.shape
    Cout = conv_w.shape[0]
    HW = H * W

    x = x_nchw.reshape(N, Cin, HW).astype(jnp.float32)
    w = conv_w.reshape(Cout, Cin).astype(jnp.float32)
    b = conv_b.reshape(Cout, 1).astype(jnp.float32)

    hw_pad = _round_up(HW, 128)
    ts = min(hw_pad, 1024)
    n_tiles = pl.cdiv(HW, ts)
    need_mask = HW % ts != 0

    vmem = int(min(48 << 20,
                   2 * Cin * ts * 4
                   + 2 * Cout * Cin * 4
                   + 2 * Cout * HW * 4
                   + _round_up(Cin, 8) * 128 * 4
                   + (8 << 20)))

    out = pl.pallas_call(
        functools.partial(_fused_kernel, hw=HW, ts=ts, inv_hw=1.0 / HW,
                          need_mask=need_mask),
        out_shape=jax.ShapeDtypeStruct((N, Cout, HW), jnp.float32),
        grid=(N, n_tiles),
        in_specs=[
            pl.BlockSpec((1, Cin, ts), lambda n, s: (n, 0, s)),
            pl.BlockSpec((Cout, Cin), lambda n, s: (0, 0)),
            pl.BlockSpec((Cout, 1), lambda n, s: (0, 0)),
        ],
        out_specs=pl.BlockSpec((1, Cout, HW), lambda n, s: (n, 0, 0)),
        scratch_shapes=[pltpu.VMEM((Cin, 128), jnp.float32)],
        compiler_params=pltpu.CompilerParams(
            dimension_semantics=("parallel", "arbitrary"),
            vmem_limit_bytes=vmem),
    )(x, w, b)

    return out.reshape(N, Cout, H, W)


# speedup vs baseline: 1.1217x; 1.0037x over previous
import functools

import jax
import jax.numpy as jnp
from jax.experimental import pallas as pl
from jax.experimental.pallas import tpu as pltpu


def _round_up(x, m):
    return (x + m - 1) // m * m


def _fused_kernel(*refs, hw, ts, k_split, inv_hw, need_mask):
    x_refs = refs[:k_split]
    w_ref, b_ref, o_ref = refs[k_split:k_split + 3]
    s = pl.program_id(1)
    nsteps = pl.num_programs(1)
    cin_k = x_refs[0].shape[1]

    if need_mask:
        pos = s * ts + jax.lax.broadcasted_iota(jnp.int32, (cin_k, ts), 1)
        mask = pos < hw

    m = jnp.zeros((w_ref.shape[0], 128), jnp.float32)
    for k, x_ref in enumerate(x_refs):
        x = x_ref[0]
        if need_mask:
            x = jnp.where(mask, x, 0.0)
        acc = x[:, 0:128]
        for j in range(1, ts // 128):
            acc = acc + x[:, j * 128:(j + 1) * 128]
        m = m + jnp.dot(w_ref[:, k * cin_k:(k + 1) * cin_k], acc,
                        preferred_element_type=jnp.float32,
                        precision=jax.lax.Precision.HIGHEST)

    y = jnp.sum(m, axis=1, keepdims=True) * inv_hw + b_ref[...]
    o_ref[...] = jnp.broadcast_to(y[None], o_ref.shape)


def _accum_kernel(*refs, hw, ts, k_split, inv_hw, need_mask):
    x_refs = refs[:k_split]
    w_ref, b_ref, o_ref, acc_ref = refs[k_split:k_split + 4]
    s = pl.program_id(1)
    cin_k = x_refs[0].shape[1]

    @pl.when(s == 0)
    def _():
        acc_ref[...] = jnp.zeros_like(acc_ref)

    if need_mask:
        pos = s * ts + jax.lax.broadcasted_iota(jnp.int32, (cin_k, ts), 1)
        mask = pos < hw

    acc_all = acc_ref[...]
    for k, x_ref in enumerate(x_refs):
        x = x_ref[0]
        if need_mask:
            x = jnp.where(mask, x, 0.0)
        acc = acc_all[k * cin_k:(k + 1) * cin_k]
        for j in range(ts // 128):
            acc = acc + x[:, j * 128:(j + 1) * 128]
        acc_ref[k * cin_k:(k + 1) * cin_k] = acc

    @pl.when(s == pl.num_programs(1) - 1)
    def _():
        m = jnp.dot(w_ref[...], acc_ref[...],
                    preferred_element_type=jnp.float32,
                    precision=jax.lax.Precision.HIGHEST)
        y = jnp.sum(m, axis=1, keepdims=True) * inv_hw + b_ref[...]
        o_ref[...] = jnp.broadcast_to(y[None], o_ref.shape)


def kernel(x_nchw, conv_w, conv_b):
    N, Cin, H, W = x_nchw.shape
    Cout = conv_w.shape[0]
    HW = H * W

    x = x_nchw.reshape(N, Cin, HW).astype(jnp.float32)
    w = conv_w.reshape(Cout, Cin).astype(jnp.float32)
    b = conv_b.reshape(Cout, 1).astype(jnp.float32)

    hw_pad = _round_up(HW, 128)
    ts = min(hw_pad, 1024)
    n_tiles = pl.cdiv(HW, ts)
    need_mask = HW % ts != 0

    k_split = 1
    for cand in (4, 2):
        if Cin % (cand * 8) == 0:
            k_split = cand
            break
    cin_k = Cin // k_split

    def _mk_idx(k):
        return lambda n, s: (n, k, s)

    x_specs = [pl.BlockSpec((1, cin_k, ts), _mk_idx(k)) for k in range(k_split)]

    vmem = int(min(56 << 20,
                   2 * Cin * ts * 4
                   + 2 * Cout * Cin * 4
                   + 2 * Cout * hw_pad * 4
                   + _round_up(Cin, 8) * 128 * 4
                   + (8 << 20)))

    if n_tiles == 1:
        body = functools.partial(_fused_kernel, hw=HW, ts=ts, k_split=k_split,
                                 inv_hw=1.0 / HW, need_mask=need_mask)
        scratch = []
    else:
        body = functools.partial(_accum_kernel, hw=HW, ts=ts, k_split=k_split,
                                 inv_hw=1.0 / HW, need_mask=need_mask)
        scratch = [pltpu.VMEM((Cin, 128), jnp.float32)]

    out = pl.pallas_call(
        body,
        out_shape=jax.ShapeDtypeStruct((N, Cout, HW), jnp.float32),
        grid=(N, n_tiles),
        in_specs=x_specs + [
            pl.BlockSpec((Cout, Cin), lambda n, s: (0, 0)),
            pl.BlockSpec((Cout, 1), lambda n, s: (0, 0)),
        ],
        out_specs=pl.BlockSpec((1, Cout, HW), lambda n, s: (n, 0, 0)),
        scratch_shapes=scratch,
        compiler_params=pltpu.CompilerParams(
            dimension_semantics=("parallel", "arbitrary"),
            vmem_limit_bytes=vmem),
    )(*([x] * k_split + [w, b]))

    return out.reshape(N, Cout, H, W)


# speedup vs baseline: 1.1613x; 1.0353x over previous
import functools

import jax
import jax.numpy as jnp
from jax.experimental import pallas as pl
from jax.experimental.pallas import tpu as pltpu


def _round_up(x, m):
    return (x + m - 1) // m * m


def _fused_kernel(x_ref, w_ref, b_ref, o_ref, acc_ref, *, hw, ts, inv_hw, need_mask):
    s = pl.program_id(1)

    @pl.when(s == 0)
    def _():
        acc_ref[...] = jnp.zeros_like(acc_ref)

    x = x_ref[0]
    if need_mask:
        pos = s * ts + jax.lax.broadcasted_iota(jnp.int32, x.shape, 1)
        x = jnp.where(pos < hw, x, 0.0)

    acc = acc_ref[...]
    for j in range(ts // 128):
        acc = acc + x[:, j * 128:(j + 1) * 128]
    acc_ref[...] = acc

    @pl.when(s == pl.num_programs(1) - 1)
    def _():
        m = jnp.dot(w_ref[...], acc_ref[...],
                    preferred_element_type=jnp.float32,
                    precision=jax.lax.Precision.DEFAULT)
        y = jnp.sum(m, axis=1, keepdims=True) * inv_hw + b_ref[...]
        o_ref[...] = jnp.broadcast_to(y[None], o_ref.shape)


def kernel(x_nchw, conv_w, conv_b):
    N, Cin, H, W = x_nchw.shape
    Cout = conv_w.shape[0]
    HW = H * W

    x = x_nchw.reshape(N, Cin, HW).astype(jnp.float32)
    w = conv_w.reshape(Cout, Cin).astype(jnp.float32)
    b = conv_b.reshape(Cout, 1).astype(jnp.float32)

    hw_pad = _round_up(HW, 128)
    ts = min(hw_pad, 1024)
    n_tiles = pl.cdiv(HW, ts)
    need_mask = HW % ts != 0

    vmem = int(min(56 << 20,
                   2 * Cin * ts * 4
                   + 2 * Cout * Cin * 4
                   + 2 * Cout * hw_pad * 4
                   + _round_up(Cin, 8) * 128 * 4
                   + (8 << 20)))

    out = pl.pallas_call(
        functools.partial(_fused_kernel, hw=HW, ts=ts, inv_hw=1.0 / HW,
                          need_mask=need_mask),
        out_shape=jax.ShapeDtypeStruct((N, Cout, HW), jnp.float32),
        grid=(N, n_tiles),
        in_specs=[
            pl.BlockSpec((1, Cin, ts), lambda n, s: (n, 0, s)),
            pl.BlockSpec((Cout, Cin), lambda n, s: (0, 0)),
            pl.BlockSpec((Cout, 1), lambda n, s: (0, 0)),
        ],
        out_specs=pl.BlockSpec((1, Cout, HW), lambda n, s: (n, 0, 0)),
        scratch_shapes=[pltpu.VMEM((Cin, 128), jnp.float32)],
        compiler_params=pltpu.CompilerParams(
            dimension_semantics=("parallel", "arbitrary"),
            vmem_limit_bytes=vmem),
    )(x, w, b)

    return out.reshape(N, Cout, H, W)


# speedup vs baseline: 1.1628x; 1.0012x over previous
import functools

import jax
import jax.numpy as jnp
from jax.experimental import pallas as pl
from jax.experimental.pallas import tpu as pltpu


def _round_up(x, m):
    return (x + m - 1) // m * m


def _fused_kernel(x_ref, w_ref, b_ref, o_ref, *, inv_hw):
    nb = x_ref.shape[0]
    hw = x_ref.shape[2]
    for i in range(nb):
        x = x_ref[i]
        acc = x[:, 0:128]
        for j in range(1, hw // 128):
            acc = acc + x[:, j * 128:(j + 1) * 128]
        m = jnp.dot(w_ref[...], acc,
                    preferred_element_type=jnp.float32,
                    precision=jax.lax.Precision.DEFAULT)
        y = jnp.sum(m, axis=1, keepdims=True) * inv_hw + b_ref[...]
        o_ref[i] = jnp.broadcast_to(y, o_ref.shape[1:])


def kernel(x_nchw, conv_w, conv_b):
    N, Cin, H, W = x_nchw.shape
    Cout = conv_w.shape[0]
    HW = H * W

    x = x_nchw.reshape(N, Cin, HW).astype(jnp.float32)
    w = conv_w.reshape(Cout, Cin).astype(jnp.float32)
    b = conv_b.reshape(Cout, 1).astype(jnp.float32)

    if HW % 128 != 0:
        pad = _round_up(HW, 128) - HW
        x = jnp.pad(x, ((0, 0), (0, 0), (0, pad)))
    hwp = x.shape[2]

    nb = 1
    for cand in (2, 1):
        if N % cand == 0 and 2 * cand * (Cin + Cout) * hwp * 4 < (40 << 20):
            nb = cand
            break

    vmem = int(min(56 << 20,
                   2 * nb * Cin * hwp * 4
                   + 2 * Cout * Cin * 4
                   + 2 * nb * Cout * hwp * 4
                   + (6 << 20)))

    out = pl.pallas_call(
        functools.partial(_fused_kernel, inv_hw=1.0 / HW),
        out_shape=jax.ShapeDtypeStruct((N, Cout, hwp), jnp.float32),
        grid=(N // nb,),
        in_specs=[
            pl.BlockSpec((nb, Cin, hwp), lambda n: (n, 0, 0)),
            pl.BlockSpec((Cout, Cin), lambda n: (0, 0)),
            pl.BlockSpec((Cout, 1), lambda n: (0, 0)),
        ],
        out_specs=pl.BlockSpec((nb, Cout, hwp), lambda n: (n, 0, 0)),
        compiler_params=pltpu.CompilerParams(
            dimension_semantics=("parallel",),
            vmem_limit_bytes=vmem),
    )(x, w, b)

    return out[:, :, :HW].reshape(N, Cout, H, W)
